# baseline (speedup 1.0000x reference)
import functools

import jax
import jax.numpy as jnp
from jax import lax
from jax.experimental import pallas as pl
from jax.experimental.pallas import tpu as pltpu
from jax.experimental.pallas import tpu_sc as plsc

NC = 2
NS = 16
NW = NC * NS

LANES = 16
SUB = 32
UNROLL = 8


def _pe_add(x_hbm, pe_hbm, out_hbm, pe_buf, x_buf):
    B = x_hbm.shape[0]
    S_D = x_hbm.shape[1]

    c = lax.axis_index("c")
    s = lax.axis_index("s")
    wid = s * NC + c

    words_w = S_D // NW
    chunk = SUB * 1024
    nsub = words_w // chunk
    base = wid * words_w

    def sub_body(j, carry):
        off = base + j * chunk
        pltpu.sync_copy(pe_hbm.at[pl.ds(off, chunk)], pe_buf)

        def batch_body(b, carry2):
            pltpu.sync_copy(x_hbm.at[b, pl.ds(off, chunk)], x_buf)

            def vec_body(i, carry3):
                o = i * (LANES * UNROLL)
                for u in range(UNROLL):
                    oo = o + u * LANES
                    v = pe_buf[pl.ds(oo, LANES)]
                    plsc.addupdate(x_buf.at[pl.ds(oo, LANES)], v)
                return carry3

            lax.fori_loop(0, chunk // (LANES * UNROLL), vec_body, 0)
            pltpu.sync_copy(x_buf, out_hbm.at[b, pl.ds(off, chunk)])
            return carry2

        lax.fori_loop(0, B, batch_body, 0)
        return carry

    lax.fori_loop(0, nsub, sub_body, 0)


def kernel(x, pe_weight):
    B, S, D = x.shape
    x2 = x.reshape(B, S * D)
    pe1 = pe_weight.reshape(-1)
    chunk = SUB * 1024

    mesh = plsc.VectorSubcoreMesh(
        core_axis_name="c", subcore_axis_name="s", num_cores=NC, num_subcores=NS
    )
    run = pl.kernel(
        _pe_add,
        out_type=jax.ShapeDtypeStruct((B, S * D), jnp.float32),
        mesh=mesh,
        scratch_types=[
            pltpu.VMEM((chunk,), jnp.float32),
            pltpu.VMEM((chunk,), jnp.float32),
        ],
    )
    out = run(x2, pe1)
    return out.reshape(B, S, D)


# speedup vs baseline: 1.1894x; 1.1894x over previous
import jax
import jax.numpy as jnp
from jax import lax
from jax.experimental import pallas as pl
from jax.experimental.pallas import tpu as pltpu
from jax.experimental.pallas import tpu_sc as plsc

NC = 2
NS = 16
NW = NC * NS

LANES = 16
SUB = 16
CHUNK = SUB * 1024
NBUF = 4
NPE = 2
UNROLL = 8


def _pe_add(x_hbm, pe_hbm, out_hbm,
            xb0, xb1, xb2, xb3, pb0, pb1,
            si0, si1, si2, si3, so0, so1, so2, so3, sp0, sp1):
    xb = [xb0, xb1, xb2, xb3]
    pb = [pb0, pb1]
    si = [si0, si1, si2, si3]
    so = [so0, so1, so2, so3]
    sp = [sp0, sp1]

    B = x_hbm.shape[0]
    S_D = x_hbm.shape[1]
    words_w = S_D // NW
    nsub = words_w // CHUNK
    nsteps = nsub * B

    c = lax.axis_index("c")
    s = lax.axis_index("s")
    wid = s * NC + c
    base = wid * words_w

    pending = {}

    def issue_in(step):
        j, b = divmod(step, B)
        k = step % NBUF
        pending[("in", step)] = pltpu.async_copy(
            x_hbm.at[b, pl.ds(base + j * CHUNK, CHUNK)], xb[k], si[k])

    def issue_out(step):
        j, b = divmod(step, B)
        k = step % NBUF
        pending[("out", step)] = pltpu.async_copy(
            xb[k], out_hbm.at[b, pl.ds(base + j * CHUNK, CHUNK)], so[k])

    def issue_pe(j):
        pending[("pe", j)] = pltpu.async_copy(
            pe_hbm.at[pl.ds(base + j * CHUNK, CHUNK)], pb[j % NPE], sp[j % NPE])

    issue_pe(0)
    issue_in(0)
    issue_in(1)

    for step in range(nsteps):
        j, b = divmod(step, B)
        k = step % NBUF
        p = j % NPE
        if b == 0:
            if j + 1 < nsub:
                issue_pe(j + 1)
            pending.pop(("pe", j)).wait()
        pending.pop(("in", step)).wait()

        x_buf = xb[k]
        pe_buf = pb[p]

        @plsc.parallel_loop(0, CHUNK, step=LANES, unroll=UNROLL)
        def _(i):
            plsc.addupdate(x_buf.at[pl.ds(i, LANES)], pe_buf[pl.ds(i, LANES)])

        issue_out(step)
        if step >= NBUF - 2:
            pending.pop(("out", step - (NBUF - 2))).wait()
        if step + 2 < nsteps:
            issue_in(step + 2)

    for step in range(nsteps - (NBUF - 2), nsteps):
        pending.pop(("out", step)).wait()


def kernel(x, pe_weight):
    B, S, D = x.shape
    x2 = x.reshape(B, S * D)
    pe1 = pe_weight.reshape(-1)

    mesh = plsc.VectorSubcoreMesh(
        core_axis_name="c", subcore_axis_name="s", num_cores=NC, num_subcores=NS
    )
    run = pl.kernel(
        _pe_add,
        out_type=jax.ShapeDtypeStruct((B, S * D), jnp.float32),
        mesh=mesh,
        scratch_types=(
            [pltpu.VMEM((CHUNK,), jnp.float32) for _ in range(NBUF)]
            + [pltpu.VMEM((CHUNK,), jnp.float32) for _ in range(NPE)]
            + [pltpu.SemaphoreType.DMA for _ in range(NBUF * 2 + NPE)]
        ),
    )
    out = run(x2, pe1)
    return out.reshape(B, S, D)


# speedup vs baseline: 3.2065x; 2.6958x over previous
import jax
import jax.numpy as jnp
from jax import lax
from jax.experimental import pallas as pl
from jax.experimental.pallas import tpu as pltpu
from jax.experimental.pallas import tpu_sc as plsc

NC = 2
NS = 16
NW = NC * NS

LANES = 16
SUB = 16
NBUF = 4
NPE = 2
UNROLL = 8


def _pe_add(x_hbm, pe_hbm, out_hbm,
            xb0, xb1, xb2, xb3, pb0, pb1,
            si0, si1, si2, si3, so0, so1, so2, so3, sp0, sp1):
    xb = [xb0, xb1, xb2, xb3]
    pb = [pb0, pb1]
    si = [si0, si1, si2, si3]
    so = [so0, so1, so2, so3]
    sp = [sp0, sp1]

    B, S, D = x_hbm.shape
    rows_w = S // NW
    nsub = rows_w // SUB
    nsteps = nsub * B
    spb = NPE * B

    c = lax.axis_index("c")
    s = lax.axis_index("s")
    wid = s * NC + c
    base = wid * rows_w

    def in_desc(j, b, k):
        return pltpu.make_async_copy(
            x_hbm.at[b, pl.ds(base + j * SUB, SUB), :], xb[k], si[k])

    def out_desc(j, b, k):
        return pltpu.make_async_copy(
            xb[k], out_hbm.at[b, pl.ds(base + j * SUB, SUB), :], so[k])

    def pe_desc(j, p):
        return pltpu.make_async_copy(
            pe_hbm.at[pl.ds(base + j * SUB, SUB), :], pb[p], sp[p])

    pe_desc(0, 0).start()
    pe_desc(1, 1).start()
    in_desc(0, 0, 0).start()
    in_desc(0, 1, 1).start()

    def body(t, carry):
        st0 = t * spb
        for u in range(spb):
            st = st0 + u
            j = st // B
            b = u % B
            k = u % NBUF
            p = u // B

            if u == B:
                @pl.when(t * NPE + 2 < nsub)
                def _():
                    pe_desc(t * NPE + 2, 0).start()
            if u == 0:
                pe_desc(j, 0).wait()
            if u == B:
                pe_desc(j, 1).wait()

            in_desc(j, b, k).wait()

            x_buf = xb[k]
            pe_buf = pb[p]

            dshift = D.bit_length() - 1

            @plsc.parallel_loop(0, SUB * D, step=LANES, unroll=UNROLL)
            def _(i):
                r = lax.shift_right_logical(i, dshift)
                col = pl.multiple_of(lax.bitwise_and(i, D - 1), LANES)
                plsc.addupdate(x_buf.at[r, pl.ds(col, LANES)],
                               pe_buf[r, pl.ds(col, LANES)])

            out_desc(j, b, k).start()

            stp = st - (NBUF - 2)
            if u >= NBUF - 2:
                up = u - (NBUF - 2)
                out_desc(stp // B, up % B, up % NBUF).wait()
            else:
                up = u - (NBUF - 2) + spb
                @pl.when(t > 0)
                def _():
                    out_desc(stp // B, up % B, up % NBUF).wait()

            stn = st + 2
            if u < spb - 2:
                un = u + 2
                in_desc(stn // B, un % B, un % NBUF).start()
            else:
                un = u + 2 - spb
                @pl.when(stn < nsteps)
                def _():
                    in_desc(stn // B, un % B, un % NBUF).start()

            if u == spb - 1:
                @pl.when(t * NPE + 3 < nsub)
                def _():
                    pe_desc(t * NPE + 3, 1).start()
        return carry

    lax.fori_loop(0, nsub // NPE, body, 0)

    for st in (nsteps - 2, nsteps - 1):
        u = (st % spb)
        out_desc(st // B, st % B, u % NBUF).wait()


def kernel(x, pe_weight):
    B, S, D = x.shape

    mesh = plsc.VectorSubcoreMesh(
        core_axis_name="c", subcore_axis_name="s", num_cores=NC, num_subcores=NS
    )
    run = pl.kernel(
        _pe_add,
        out_type=jax.ShapeDtypeStruct((B, S, D), jnp.float32),
        mesh=mesh,
        scratch_types=(
            [pltpu.VMEM((SUB, D), jnp.float32) for _ in range(NBUF)]
            + [pltpu.VMEM((SUB, D), jnp.float32) for _ in range(NPE)]
            + [pltpu.SemaphoreType.DMA for _ in range(NBUF * 2 + NPE)]
        ),
    )
    return run(x, pe_weight)


# speedup vs baseline: 3.4635x; 1.0802x over previous
import jax
import jax.numpy as jnp
from jax import lax
from jax.experimental import pallas as pl
from jax.experimental.pallas import tpu as pltpu
from jax.experimental.pallas import tpu_sc as plsc

NC = 2
NS = 16
NW = NC * NS

LANES = 16
SUB = 8
NBUF = 8
NPE = 2
PD = 4
UNROLL = 8


def _pe_add(x_hbm, pe_hbm, out_hbm,
            xb0, xb1, xb2, xb3, xb4, xb5, xb6, xb7, pb0, pb1,
            si0, si1, si2, si3, si4, si5, si6, si7,
            so0, so1, so2, so3, so4, so5, so6, so7, sp0, sp1):
    xb = [xb0, xb1, xb2, xb3, xb4, xb5, xb6, xb7]
    pb = [pb0, pb1]
    si = [si0, si1, si2, si3, si4, si5, si6, si7]
    so = [so0, so1, so2, so3, so4, so5, so6, so7]
    sp = [sp0, sp1]

    B, S, D = x_hbm.shape
    rows_w = S // NW
    nsub = rows_w // SUB
    nsteps = nsub * B
    spb = NPE * B
    nbody = nsteps // spb

    c = lax.axis_index("c")
    s = lax.axis_index("s")
    wid = s * NC + c
    base = wid * rows_w

    def in_desc(j, b, k):
        return pltpu.make_async_copy(
            x_hbm.at[b, pl.ds(base + j * SUB, SUB), :], xb[k], si[k])

    def out_desc(j, b, k):
        return pltpu.make_async_copy(
            xb[k], out_hbm.at[b, pl.ds(base + j * SUB, SUB), :], so[k])

    def pe_desc(j, p):
        return pltpu.make_async_copy(
            pe_hbm.at[pl.ds(base + j * SUB, SUB), :], pb[p], sp[p])

    pe_desc(0, 0).start()
    pe_desc(1, 1).start()
    for st in range(PD):
        in_desc(st // B, st % B, st % NBUF).start()

    dshift = D.bit_length() - 1

    def body(t, carry):
        st0 = t * spb
        for u in range(spb):
            st = st0 + u
            j = st // B
            b = u % B
            k = u % NBUF
            p = u // B

            if u == 0:
                pe_desc(j, 0).wait()
            if u == B:
                @pl.when(t * NPE + 2 < nsub)
                def _():
                    pe_desc(t * NPE + 2, 0).start()
                pe_desc(j, 1).wait()

            in_desc(j, b, k).wait()

            x_buf = xb[k]
            pe_buf = pb[p]

            @plsc.parallel_loop(0, SUB * D, step=LANES, unroll=UNROLL)
            def _(i):
                r = lax.shift_right_logical(i, dshift)
                col = pl.multiple_of(lax.bitwise_and(i, D - 1), LANES)
                plsc.addupdate(x_buf.at[r, pl.ds(col, LANES)],
                               pe_buf[r, pl.ds(col, LANES)])

            out_desc(j, b, k).start()

            stp = st - PD
            stn = st + PD
            if u >= PD:
                up = u - PD
                out_desc(stp // B, up % B, up % NBUF).wait()
                un = u - PD
                @pl.when(t < nbody - 1)
                def _():
                    in_desc(stn // B, un % B, un % NBUF).start()
            else:
                up = u + spb - PD
                @pl.when(t > 0)
                def _():
                    out_desc(stp // B, up % B, up % NBUF).wait()
                un = u + PD
                in_desc(stn // B, un % B, un % NBUF).start()

            if u == spb - 1:
                @pl.when(t * NPE + 3 < nsub)
                def _():
                    pe_desc(t * NPE + 3, 1).start()
        return carry

    lax.fori_loop(0, nbody, body, 0)

    for st in range(nsteps - PD, nsteps):
        out_desc(st // B, st % B, (st % spb) % NBUF).wait()


def kernel(x, pe_weight):
    B, S, D = x.shape

    mesh = plsc.VectorSubcoreMesh(
        core_axis_name="c", subcore_axis_name="s", num_cores=NC, num_subcores=NS
    )
    run = pl.kernel(
        _pe_add,
        out_type=jax.ShapeDtypeStruct((B, S, D), jnp.float32),
        mesh=mesh,
        scratch_types=(
            [pltpu.VMEM((SUB, D), jnp.float32) for _ in range(NBUF)]
            + [pltpu.VMEM((SUB, D), jnp.float32) for _ in range(NPE)]
            + [pltpu.SemaphoreType.DMA for _ in range(NBUF * 2 + NPE)]
        ),
    )
    return run(x, pe_weight)
